# Initial kernel scaffold; baseline (speedup 1.0000x reference)
import functools

import jax
import jax.numpy as jnp
from jax import lax
from jax.experimental import pallas as pl
from jax.experimental.pallas import tpu as pltpu
from jax.experimental.pallas import tpu_sc as plsc

NUM_ROWS = 16384
NUM_COLS = 26
DIM = 32

B_TOTAL = NUM_ROWS * NUM_COLS
NC = 2
NS = 16
NW = NC * NS
B_PER_W = B_TOTAL // NW
CHUNK = 128
N_CHUNKS = B_PER_W // CHUNK

_mesh = plsc.VectorSubcoreMesh(core_axis_name="c", subcore_axis_name="s")


@functools.partial(
    pl.kernel,
    out_type=jax.ShapeDtypeStruct((B_TOTAL, DIM), jnp.float32),
    mesh=_mesh,
    scratch_types=[
        pltpu.VMEM((N_CHUNKS, CHUNK), jnp.int32),
        pltpu.VMEM((CHUNK, DIM), jnp.float32),
        pltpu.SemaphoreType.DMA,
    ],
)
def _emb_lookup(idx_hbm, table_hbm, out_hbm, idx_v, rows_v, sem):
    wid = lax.axis_index("s") * NC + lax.axis_index("c")
    base = wid * B_PER_W
    pltpu.sync_copy(idx_hbm.at[wid], idx_v)

    def body(i, carry):
        pltpu.async_copy(table_hbm.at[idx_v.at[i]], rows_v, sem).wait()
        pltpu.sync_copy(rows_v, out_hbm.at[pl.ds(base + i * CHUNK, CHUNK)])
        return carry

    lax.fori_loop(0, N_CHUNKS, body, 0)


def kernel(input, weight):
    idx = input.reshape(NW, N_CHUNKS, CHUNK)
    out = _emb_lookup(idx, weight)
    return out.reshape(NUM_ROWS, NUM_COLS, DIM)


# speedup vs baseline: 1.4366x; 1.4366x over previous
import functools

import jax
import jax.numpy as jnp
from jax import lax
from jax.experimental import pallas as pl
from jax.experimental.pallas import tpu as pltpu
from jax.experimental.pallas import tpu_sc as plsc

NUM_ROWS = 16384
NUM_COLS = 26
DIM = 32

B_TOTAL = NUM_ROWS * NUM_COLS
NC = 2
NS = 16
NW = NC * NS
B_PER_W = B_TOTAL // NW
CHUNK = 128
N_CHUNKS = B_PER_W // CHUNK

_mesh = plsc.VectorSubcoreMesh(core_axis_name="c", subcore_axis_name="s")


@functools.partial(
    pl.kernel,
    out_type=jax.ShapeDtypeStruct((B_TOTAL, DIM), jnp.float32),
    mesh=_mesh,
    scratch_types=[
        pltpu.VMEM((N_CHUNKS, CHUNK), jnp.int32),
        pltpu.VMEM((CHUNK, DIM), jnp.float32),
        pltpu.SemaphoreType.DMA,
    ],
    compiler_params=pltpu.CompilerParams(use_tc_tiling_on_sc=False),
)
def _emb_lookup(idx_hbm, table_hbm, out_hbm, idx_v, rows_v, sem):
    wid = lax.axis_index("s") * NC + lax.axis_index("c")
    base = wid * B_PER_W
    pltpu.sync_copy(idx_hbm.at[wid], idx_v)

    def body(i, carry):
        pltpu.async_copy(table_hbm.at[idx_v.at[i]], rows_v, sem).wait()
        pltpu.sync_copy(rows_v, out_hbm.at[pl.ds(base + i * CHUNK, CHUNK)])
        return carry

    lax.fori_loop(0, N_CHUNKS, body, 0)


def kernel(input, weight):
    idx = input.reshape(NW, N_CHUNKS, CHUNK)
    out = _emb_lookup(idx, weight)
    return out.reshape(NUM_ROWS, NUM_COLS, DIM)


# speedup vs baseline: 1.5536x; 1.0814x over previous
import functools

import jax
import jax.numpy as jnp
from jax import lax
from jax.experimental import pallas as pl
from jax.experimental.pallas import tpu as pltpu
from jax.experimental.pallas import tpu_sc as plsc

NUM_ROWS = 16384
NUM_COLS = 26
DIM = 32

B_TOTAL = NUM_ROWS * NUM_COLS
NC = 2
NS = 16
NW = NC * NS
B_PER_W = B_TOTAL // NW
CHUNK = 128
N_CHUNKS = B_PER_W // CHUNK
NBUF = 4
N_GROUPS = N_CHUNKS // NBUF
G2 = N_GROUPS // 2

_mesh = plsc.VectorSubcoreMesh(core_axis_name="c", subcore_axis_name="s")


@functools.partial(
    pl.kernel,
    out_type=jax.ShapeDtypeStruct((B_TOTAL, DIM), jnp.float32),
    mesh=_mesh,
    scratch_types=[
        pltpu.VMEM((N_CHUNKS, CHUNK), jnp.int32),
        pltpu.VMEM((NBUF, CHUNK, DIM), jnp.float32),
        pltpu.VMEM((NBUF, CHUNK, DIM), jnp.float32),
        pltpu.SemaphoreType.DMA,
        pltpu.SemaphoreType.DMA,
        pltpu.SemaphoreType.DMA,
        pltpu.SemaphoreType.DMA,
    ],
    compiler_params=pltpu.CompilerParams(use_tc_tiling_on_sc=False),
)
def _emb_lookup(idx_hbm, table_hbm, out_hbm, idx_v, buf_a, buf_b,
                gsem_a, gsem_b, ssem_a, ssem_b):
    wid = lax.axis_index("s") * NC + lax.axis_index("c")
    base = wid * B_PER_W
    pltpu.sync_copy(idx_hbm.at[wid], idx_v)

    def fire_gathers(g, buf, gsem):
        for b in range(NBUF):
            i = g * NBUF + b
            pltpu.async_copy(table_hbm.at[idx_v.at[i]], buf.at[b], gsem)

    def drain_gathers(g, buf, gsem):
        for b in range(NBUF):
            i = g * NBUF + b
            pltpu.make_async_copy(table_hbm.at[idx_v.at[i]], buf.at[b],
                                  gsem).wait()

    def fire_stores(g, buf, ssem):
        for b in range(NBUF):
            i = g * NBUF + b
            pltpu.async_copy(buf.at[b],
                             out_hbm.at[pl.ds(base + i * CHUNK, CHUNK)], ssem)

    def drain_stores(buf, ssem):
        for b in range(NBUF):
            pltpu.make_async_copy(buf.at[b], out_hbm.at[pl.ds(base, CHUNK)],
                                  ssem).wait()

    fire_gathers(0, buf_a, gsem_a)

    def body(g2, carry):
        ga = 2 * g2
        gb = ga + 1
        drain_gathers(ga, buf_a, gsem_a)

        @pl.when(g2 > 0)
        def _():
            drain_stores(buf_b, ssem_b)

        fire_gathers(gb, buf_b, gsem_b)
        fire_stores(ga, buf_a, ssem_a)
        drain_gathers(gb, buf_b, gsem_b)
        drain_stores(buf_a, ssem_a)

        @pl.when(g2 < G2 - 1)
        def _():
            fire_gathers(ga + 2, buf_a, gsem_a)

        fire_stores(gb, buf_b, ssem_b)
        return carry

    lax.fori_loop(0, G2, body, 0)
    drain_stores(buf_b, ssem_b)


def kernel(input, weight):
    idx = input.reshape(NW, N_CHUNKS, CHUNK)
    out = _emb_lookup(idx, weight)
    return out.reshape(NUM_ROWS, NUM_COLS, DIM)


# speedup vs baseline: 1.6454x; 1.0591x over previous
import functools

import jax
import jax.numpy as jnp
from jax import lax
from jax.experimental import pallas as pl
from jax.experimental.pallas import tpu as pltpu
from jax.experimental.pallas import tpu_sc as plsc

NUM_ROWS = 16384
NUM_COLS = 26
DIM = 32

B_TOTAL = NUM_ROWS * NUM_COLS
NUM_EMB = 1000000
NC = 2
NS = 16
NW = NC * NS
B_PER_W = B_TOTAL // NW
CHUNK = 128
N_CHUNKS = B_PER_W // CHUNK
NBUF = 4
N_GROUPS = N_CHUNKS // NBUF
G2 = N_GROUPS // 2

_mesh = plsc.VectorSubcoreMesh(core_axis_name="c", subcore_axis_name="s")


@functools.partial(
    pl.kernel,
    out_type=jax.ShapeDtypeStruct((B_TOTAL, DIM), jnp.float32),
    mesh=_mesh,
    scratch_types=[
        pltpu.VMEM((N_CHUNKS, CHUNK), jnp.int32),
        pltpu.VMEM((NBUF, CHUNK, DIM), jnp.float32),
        pltpu.VMEM((NBUF, CHUNK, DIM), jnp.float32),
        pltpu.SemaphoreType.DMA,
        pltpu.SemaphoreType.DMA,
        pltpu.SemaphoreType.DMA,
        pltpu.SemaphoreType.DMA,
    ],
    compiler_params=pltpu.CompilerParams(use_tc_tiling_on_sc=False),
)
def _emb_lookup(idx_hbm, table_hbm, out_hbm, idx_v, buf_a, buf_b,
                gsem_a, gsem_b, ssem_a, ssem_b):
    wid = lax.axis_index("s") * NC + lax.axis_index("c")
    base = wid * B_PER_W
    pltpu.sync_copy(idx_hbm.at[wid], idx_v)

    def fire_gathers(g, buf, gsem):
        for b in range(NBUF):
            i = g * NBUF + b
            pltpu.async_copy(table_hbm.at[idx_v.at[i]], buf.at[b], gsem)

    def drain_gathers(g, buf, gsem):
        for b in range(NBUF):
            i = g * NBUF + b
            pltpu.make_async_copy(table_hbm.at[idx_v.at[i]], buf.at[b],
                                  gsem).wait()

    def fire_stores(g, buf, ssem):
        for b in range(NBUF):
            i = g * NBUF + b
            pltpu.async_copy(buf.at[b],
                             out_hbm.at[pl.ds(base + i * CHUNK, CHUNK)], ssem)

    def drain_stores(buf, ssem):
        for b in range(NBUF):
            pltpu.make_async_copy(buf.at[b], out_hbm.at[pl.ds(base, CHUNK)],
                                  ssem).wait()

    fire_gathers(0, buf_a, gsem_a)

    def body(g2, carry):
        ga = 2 * g2
        gb = ga + 1
        drain_gathers(ga, buf_a, gsem_a)

        @pl.when(g2 > 0)
        def _():
            drain_stores(buf_b, ssem_b)

        fire_gathers(gb, buf_b, gsem_b)
        fire_stores(ga, buf_a, ssem_a)
        drain_gathers(gb, buf_b, gsem_b)
        drain_stores(buf_a, ssem_a)

        @pl.when(g2 < G2 - 1)
        def _():
            fire_gathers(ga + 2, buf_a, gsem_a)

        fire_stores(gb, buf_b, ssem_b)
        return carry

    lax.fori_loop(0, G2, body, 0)
    drain_stores(buf_b, ssem_b)


def kernel(input, weight):
    idx = jnp.swapaxes(input, 0, 1).reshape(NW, N_CHUNKS, CHUNK)
    out = _emb_lookup(idx, weight)
    out3 = out.reshape(NUM_COLS, NUM_ROWS, DIM)
    return jnp.swapaxes(out3, 0, 1)
